# Initial kernel scaffold; baseline (speedup 1.0000x reference)
import jax
import jax.numpy as jnp
from jax.experimental import pallas as pl


def kernel(node_feat, dist, edge_index, W_edge, W_node):
    raise NotImplementedError("write your pallas kernel here")


# speedup vs baseline: 5.9262x; 5.9262x over previous
import functools

import jax
import jax.numpy as jnp
from jax import lax
from jax.experimental import pallas as pl
from jax.experimental.pallas import tpu as pltpu
from jax.experimental.pallas import tpu_sc as plsc

LANES = 16
ROW_BLK = 512
CH = 2048
GRP = 64
DRAIN = 8
BF_INF2 = 0x7F807F80


def _leaky(x):
    return jnp.where(x >= 0, x, 0.01 * x)


def _dotT(x, w):
    return lax.dot_general(
        x, w, dimension_numbers=(((1,), (1,)), ((), ())),
        preferred_element_type=jnp.float32,
        precision=lax.Precision.HIGHEST)


def _proj_body(x_ref, w_ref, o_ref):
    bf = _dotT(x_ref[:], w_ref[:]).astype(jnp.bfloat16)
    h = bf.shape[1] // 2
    lo = lax.bitcast_convert_type(bf[:, :h], jnp.uint16).astype(jnp.int32)
    hi = lax.bitcast_convert_type(bf[:, h:], jnp.uint16).astype(jnp.int32)
    o_ref[:] = lo | (hi << 16)


def _proj(nf_pad, W):
    NP, D = nf_pad.shape
    O = W.shape[0]
    grid = (-(-NP // ROW_BLK),)
    return pl.pallas_call(
        _proj_body,
        grid=grid,
        in_specs=[
            pl.BlockSpec((ROW_BLK, D), lambda i: (i, 0)),
            pl.BlockSpec((O, D), lambda i: (0, 0)),
        ],
        out_specs=pl.BlockSpec((ROW_BLK, O // 2), lambda i: (i, 0)),
        out_shape=jax.ShapeDtypeStruct((NP, O // 2), jnp.int32),
    )(nf_pad, W)


def _final_body(nf_ref, m_ref, wv_ref, wn1_ref, wn2_ref, o_ref):
    nf = nf_ref[:]
    m32 = m_ref[:]
    m_lo = lax.bitcast_convert_type(
        (m32 & 0xFFFF).astype(jnp.uint16), jnp.bfloat16).astype(jnp.float32)
    m_hi = lax.bitcast_convert_type(
        lax.shift_right_logical(m32, 16).astype(jnp.uint16),
        jnp.bfloat16).astype(jnp.float32)
    m = jnp.concatenate([m_lo, m_hi], axis=1)
    t = _dotT(nf, wv_ref[:]) + m
    aggr = jnp.where(jnp.isfinite(t), _leaky(t), 0.0)
    o_ref[:] = _leaky(_dotT(nf, wn1_ref[:]) + _dotT(aggr, wn2_ref[:]))


def _final(nf_pad, m2d, Wv, Wn1, Wn2):
    NP, D = nf_pad.shape
    O = Wv.shape[0]
    grid = (-(-NP // ROW_BLK),)
    return pl.pallas_call(
        _final_body,
        grid=grid,
        in_specs=[
            pl.BlockSpec((ROW_BLK, D), lambda i: (i, 0)),
            pl.BlockSpec((ROW_BLK, O // 2), lambda i: (i, 0)),
            pl.BlockSpec((O, D), lambda i: (0, 0)),
            pl.BlockSpec((O, D), lambda i: (0, 0)),
            pl.BlockSpec((O, O), lambda i: (0, 0)),
        ],
        out_specs=pl.BlockSpec((ROW_BLK, O), lambda i: (i, 0)),
        out_shape=jax.ShapeDtypeStruct((NP, O), jnp.float32),
    )(nf_pad, m2d, Wv, Wn1, Wn2)


def _segmin_call(A32, src, dst, dist, wd32, NP, D, NW, NPW):
    EP = src.shape[0]
    DW = D // 2
    NV = DW // LANES
    CAP = DRAIN * CH + 2 * GRP
    mesh = plsc.VectorSubcoreMesh(core_axis_name="c", subcore_axis_name="s")
    info = plsc.get_sparse_core_info()
    NC = info.num_cores

    @functools.partial(
        pl.kernel, mesh=mesh,
        compiler_params=pltpu.CompilerParams(needs_layout_passes=False),
        out_type=jax.ShapeDtypeStruct((NP * DW,), jnp.int32),
        scratch_types=[
            pltpu.VMEM(((NPW + 1) * DW,), jnp.int32),
            pltpu.VMEM((2 * CH,), jnp.int32),
            pltpu.VMEM((2 * CH,), jnp.int32),
            pltpu.VMEM((2 * CH,), jnp.float32),
            pltpu.VMEM((CAP,), jnp.int32),
            pltpu.VMEM((CAP,), jnp.int32),
            pltpu.VMEM((CAP,), jnp.float32),
            pltpu.VMEM((2, GRP, DW), jnp.int32),
            pltpu.VMEM((DW,), jnp.int32),
            pltpu.VMEM((CH // LANES + LANES,), jnp.int32),
            pltpu.VMEM((CH // LANES + LANES,), jnp.int32),
            pltpu.SemaphoreType.DMA,
            pltpu.SemaphoreType.DMA,
            pltpu.SemaphoreType.DMA,
        ],
    )
    def seg_kernel(A_hbm, src_hbm, dst_hbm, dist_hbm, wd_hbm, m_hbm,
                   acc, dstb, srcb, distb, cdst, cidx, cdist, rows, wdv,
                   baseb, wlist, sg0, sg1, semc):
        wid = lax.axis_index("s") * NC + lax.axis_index("c")
        lo = wid * NPW

        pltpu.sync_copy(wd_hbm, wdv)

        inf16 = jnp.full((LANES,), BF_INF2, dtype=jnp.int32)

        def init_row(r, _):
            for v in range(NV):
                acc[pl.ds(r * DW + v * LANES, LANES)] = inf16
            return 0
        lax.fori_loop(0, NPW + 1, init_row, 0)

        def start_gather(g, par, gsem):
            pltpu.async_copy(A_hbm.at[cidx.at[pl.ds(g * GRP, GRP)]],
                             rows.at[par], gsem)

        def wait_gather(g, par, gsem):
            pltpu.make_async_copy(A_hbm.at[cidx.at[pl.ds(g * GRP, GRP)]],
                                  rows.at[par], gsem).wait()

        iot0 = lax.iota(jnp.int32, LANES) * 0

        def accum_group(base, par):
            def edge(j, _):
                dl = cdst[pl.ds(base + j, LANES)][0]
                dsp = cdist[pl.ds(base + j, LANES)][iot0]
                dib = plsc.pack(dsp, dsp, format=plsc.PackFormat.INTERLEAVED)
                off = dl * DW

                @plsc.parallel_loop(0, NV, 1, unroll=NV)
                def _(v):
                    sl = pl.ds(off + v * LANES, LANES)
                    rw = plsc.bitcast(rows[par, j, pl.ds(v * LANES, LANES)],
                                      jnp.bfloat16)
                    wv = plsc.bitcast(wdv[pl.ds(v * LANES, LANES)],
                                      jnp.bfloat16)
                    av = plsc.bitcast(acc[sl], jnp.bfloat16)
                    val = jnp.minimum(av, rw + dib * wv)
                    acc[sl] = plsc.bitcast(val, jnp.int32)
                return 0
            lax.fori_loop(0, GRP, edge, 0)

        def run_groups(ngroups):
            @pl.when(ngroups > 0)
            def _():
                start_gather(0, 0, sg0)

            @pl.when(ngroups > 1)
            def _():
                start_gather(1, 1, sg1)

            npairs = ngroups // 2

            def pair(k, _):
                g0 = k * 2
                wait_gather(g0, 0, sg0)
                accum_group(g0 * GRP, 0)

                @pl.when(g0 + 2 < ngroups)
                def _():
                    start_gather(g0 + 2, 0, sg0)

                g1 = g0 + 1
                wait_gather(g1, 1, sg1)
                accum_group(g1 * GRP, 1)

                @pl.when(g1 + 2 < ngroups)
                def _():
                    start_gather(g1 + 2, 1, sg1)
                return 0
            lax.fori_loop(0, npairs, pair, 0)

            @pl.when(npairs * 2 < ngroups)
            def _():
                g = npairs * 2
                wait_gather(g, 0, sg0)
                accum_group(g * GRP, 0)

        NCH = EP // CH

        def load_chunk(c, par):
            e0 = c * CH
            b = pl.ds(par * CH, CH)
            pltpu.async_copy(dst_hbm.at[pl.ds(e0, CH)], dstb.at[b], semc)
            pltpu.async_copy(src_hbm.at[pl.ds(e0, CH)], srcb.at[b], semc)
            pltpu.async_copy(dist_hbm.at[pl.ds(e0, CH)], distb.at[b], semc)

        def wait_chunk(c, par):
            e0 = c * CH
            b = pl.ds(par * CH, CH)
            pltpu.make_async_copy(dst_hbm.at[pl.ds(e0, CH)], dstb.at[b],
                                  semc).wait()
            pltpu.make_async_copy(src_hbm.at[pl.ds(e0, CH)], srcb.at[b],
                                  semc).wait()
            pltpu.make_async_copy(dist_hbm.at[pl.ds(e0, CH)], distb.at[b],
                                  semc).wait()

        load_chunk(0, 0)

        lane15 = jnp.full((LANES,), LANES - 1, dtype=jnp.int32)

        def chunk(c, count_v):
            par = lax.rem(c, jnp.int32(2))
            wait_chunk(c, par)

            @pl.when(c + 1 < NCH)
            def _():
                load_chunk(c + 1, 1 - par)

            NS = CH // LANES
            iot = lax.iota(jnp.int32, LANES)
            l0 = iot == 0

            @plsc.parallel_loop(0, NS, 1, unroll=8)
            def _(i):
                s = pl.ds(par * CH + i * LANES, LANES)
                d16 = dstb[s]
                msk = (d16 >= lo) & (d16 < lo + NPW)
                pop = plsc.all_reduce_population_count(msk)
                plsc.store_scatter(baseb, [iot * 0 + i], pop, mask=l0)

            def prefix(k, carry):
                run_v, nz_v = carry
                sl = pl.ds(k * LANES, LANES)
                v = baseb[sl]
                csum = plsc.cumsum(v)
                baseb[sl] = run_v + csum - v
                nmsk = v > 0
                ncs = plsc.cumsum(jnp.where(nmsk, 1, 0))
                plsc.store_scatter(wlist, [nz_v + ncs - 1], k * LANES + iot,
                                   mask=nmsk)
                return run_v + csum[lane15], nz_v + ncs[lane15]
            count_v, nz_v = lax.fori_loop(
                0, NS // LANES, prefix,
                (count_v, jnp.zeros((LANES,), jnp.int32)))

            nz = nz_v[0]
            nzp = ((nz + 3) // 4) * 4

            @pl.when(nz % 4 != 0)
            def _():
                wlist[pl.ds(nz, LANES)] = jnp.zeros((LANES,), jnp.int32)

            @plsc.parallel_loop(0, nzp, 1, unroll=4)
            def _(k):
                i = wlist[pl.ds(k, LANES)][0]
                bvec = baseb[pl.ds(i, LANES)]
                base = bvec[iot * 0]
                s = pl.ds(par * CH + i * LANES, LANES)
                d16 = dstb[s]
                msk = (d16 >= lo) & (d16 < lo + NPW)
                csum = plsc.cumsum(jnp.where(msk, 1, 0))
                p = base + csum - 1
                plsc.store_scatter(cdst, [p], d16 - lo, mask=msk)
                plsc.store_scatter(cidx, [p], srcb[s], mask=msk)
                plsc.store_scatter(cdist, [p], distb[s], mask=msk)

            count = count_v[0]
            do_drain = (lax.rem(c, jnp.int32(DRAIN)) == DRAIN - 1) \
                | (c == NCH - 1)
            ngroups = jnp.where(do_drain, count // GRP, 0)
            run_groups(ngroups)

            @pl.when(ngroups > 0)
            def _():
                for k in range(GRP // LANES):
                    t = pl.ds(ngroups * GRP + k * LANES, LANES)
                    h = pl.ds(k * LANES, LANES)
                    cdst[h] = cdst[t]
                    cidx[h] = cidx[t]
                    cdist[h] = cdist[t]

            return jnp.full((LANES,), count - ngroups * GRP, dtype=jnp.int32)

        rem_v = lax.fori_loop(0, EP // CH, chunk,
                              jnp.zeros((LANES,), jnp.int32))
        rem = rem_v[0]

        @pl.when(rem > 0)
        def _():
            for k in range(GRP // LANES):
                p = pl.ds(rem + k * LANES, LANES)
                cidx[p] = jnp.zeros((LANES,), jnp.int32)
                cdst[p] = jnp.full((LANES,), NPW, dtype=jnp.int32)
                cdist[p] = jnp.zeros((LANES,), jnp.float32)
            start_gather(0, 0, sg0)
            wait_gather(0, 0, sg0)
            accum_group(0, 0)

        pltpu.sync_copy(acc.at[pl.ds(0, NPW * DW)],
                        m_hbm.at[pl.ds(lo * DW, NPW * DW)])

    return seg_kernel(A32, src, dst, dist, wd32)


def kernel(node_feat, dist, edge_index, W_edge, W_node):
    N, D = node_feat.shape
    O = W_edge.shape[0]
    E = dist.shape[0]

    Wu = W_edge[:, :D]
    wd = W_edge[:, D]
    Wv = W_edge[:, D + 1:]
    Wn1 = W_node[:, :D]
    Wn2 = W_node[:, D:]

    info = plsc.get_sparse_core_info()
    NW = info.num_cores * info.num_subcores
    NPW = -(-N // (NW * 64)) * 64
    NP = NW * NPW
    NP = -(-NP // ROW_BLK) * ROW_BLK
    NPW = NP // NW

    src = edge_index[0]
    dst = edge_index[1]
    EP = -(-E // CH) * CH
    if EP != E:
        src = jnp.pad(src, (0, EP - E))
        dist = jnp.pad(dist, (0, EP - E))
        dst = jnp.pad(dst, (0, EP - E), constant_values=-1)

    A32 = _proj(node_feat, Wu)
    wd16 = wd.astype(jnp.bfloat16)
    wd_lo = lax.bitcast_convert_type(wd16[:O // 2], jnp.uint16
                                     ).astype(jnp.int32)
    wd_hi = lax.bitcast_convert_type(wd16[O // 2:], jnp.uint16
                                     ).astype(jnp.int32)
    wd32 = wd_lo | (wd_hi << 16)
    m_flat = _segmin_call(A32, src, dst, dist, wd32, NP, O, NW, NPW)
    out = _final(node_feat, m_flat.reshape(NP, O // 2), Wv, Wn1, Wn2)
    return out


# speedup vs baseline: 6.1504x; 1.0378x over previous
import functools

import jax
import jax.numpy as jnp
from jax import lax
from jax.experimental import pallas as pl
from jax.experimental.pallas import tpu as pltpu
from jax.experimental.pallas import tpu_sc as plsc

LANES = 16
ROW_BLK = 512
CH = 2048
GRP = 32
DRAIN = 8
BF_INF2 = 0x7F807F80


def _leaky(x):
    return jnp.where(x >= 0, x, 0.01 * x)


def _dotT(x, w):
    return lax.dot_general(
        x, w, dimension_numbers=(((1,), (1,)), ((), ())),
        preferred_element_type=jnp.float32,
        precision=lax.Precision.HIGHEST)


def _proj_body(x_ref, w_ref, o_ref):
    bf = _dotT(x_ref[:], w_ref[:]).astype(jnp.bfloat16)
    h = bf.shape[1] // 2
    lo = lax.bitcast_convert_type(bf[:, :h], jnp.uint16).astype(jnp.int32)
    hi = lax.bitcast_convert_type(bf[:, h:], jnp.uint16).astype(jnp.int32)
    o_ref[:] = lo | (hi << 16)


def _proj(nf_pad, W):
    NP, D = nf_pad.shape
    O = W.shape[0]
    grid = (-(-NP // ROW_BLK),)
    return pl.pallas_call(
        _proj_body,
        grid=grid,
        in_specs=[
            pl.BlockSpec((ROW_BLK, D), lambda i: (i, 0)),
            pl.BlockSpec((O, D), lambda i: (0, 0)),
        ],
        out_specs=pl.BlockSpec((ROW_BLK, O // 2), lambda i: (i, 0)),
        out_shape=jax.ShapeDtypeStruct((NP, O // 2), jnp.int32),
    )(nf_pad, W)


def _final_body(nf_ref, m_ref, wv_ref, wn1_ref, wn2_ref, o_ref):
    nf = nf_ref[:]
    m32 = m_ref[:]
    m_lo = lax.bitcast_convert_type(
        (m32 & 0xFFFF).astype(jnp.uint16), jnp.bfloat16).astype(jnp.float32)
    m_hi = lax.bitcast_convert_type(
        lax.shift_right_logical(m32, 16).astype(jnp.uint16),
        jnp.bfloat16).astype(jnp.float32)
    m = jnp.concatenate([m_lo, m_hi], axis=1)
    t = _dotT(nf, wv_ref[:]) + m
    aggr = jnp.where(jnp.isfinite(t), _leaky(t), 0.0)
    o_ref[:] = _leaky(_dotT(nf, wn1_ref[:]) + _dotT(aggr, wn2_ref[:]))


def _final(nf_pad, m2d, Wv, Wn1, Wn2):
    NP, D = nf_pad.shape
    O = Wv.shape[0]
    grid = (-(-NP // ROW_BLK),)
    return pl.pallas_call(
        _final_body,
        grid=grid,
        in_specs=[
            pl.BlockSpec((ROW_BLK, D), lambda i: (i, 0)),
            pl.BlockSpec((ROW_BLK, O // 2), lambda i: (i, 0)),
            pl.BlockSpec((O, D), lambda i: (0, 0)),
            pl.BlockSpec((O, D), lambda i: (0, 0)),
            pl.BlockSpec((O, O), lambda i: (0, 0)),
        ],
        out_specs=pl.BlockSpec((ROW_BLK, O), lambda i: (i, 0)),
        out_shape=jax.ShapeDtypeStruct((NP, O), jnp.float32),
    )(nf_pad, m2d, Wv, Wn1, Wn2)


def _segmin_call(A32, src, dst, dist, wd32, NP, D, NW, NPW):
    EP = src.shape[0]
    DW = D // 2
    NV = DW // LANES
    CAP = DRAIN * CH + 2 * GRP
    mesh = plsc.VectorSubcoreMesh(core_axis_name="c", subcore_axis_name="s")
    info = plsc.get_sparse_core_info()
    NC = info.num_cores

    @functools.partial(
        pl.kernel, mesh=mesh,
        compiler_params=pltpu.CompilerParams(needs_layout_passes=False),
        out_type=jax.ShapeDtypeStruct((NP * DW,), jnp.int32),
        scratch_types=[
            pltpu.VMEM(((NPW + 1) * DW,), jnp.int32),
            pltpu.VMEM((2 * CH,), jnp.int32),
            pltpu.VMEM((2 * CH,), jnp.int32),
            pltpu.VMEM((2 * CH,), jnp.float32),
            pltpu.VMEM((CAP,), jnp.int32),
            pltpu.VMEM((CAP,), jnp.int32),
            pltpu.VMEM((CAP,), jnp.float32),
            pltpu.VMEM((2, GRP, DW), jnp.int32),
            pltpu.VMEM((DW,), jnp.int32),
            pltpu.VMEM((CH // LANES + LANES,), jnp.int32),
            pltpu.VMEM((CH // LANES + LANES,), jnp.int32),
            pltpu.SemaphoreType.DMA,
            pltpu.SemaphoreType.DMA,
            pltpu.SemaphoreType.DMA,
        ],
    )
    def seg_kernel(A_hbm, src_hbm, dst_hbm, dist_hbm, wd_hbm, m_hbm,
                   acc, dstb, srcb, distb, cdst, cidx, cdist, rows, wdv,
                   baseb, wlist, sg0, sg1, semc):
        wid = lax.axis_index("s") * NC + lax.axis_index("c")
        lo = wid * NPW

        pltpu.sync_copy(wd_hbm, wdv)

        inf16 = jnp.full((LANES,), BF_INF2, dtype=jnp.int32)

        def init_row(r, _):
            for v in range(NV):
                acc[pl.ds(r * DW + v * LANES, LANES)] = inf16
            return 0
        lax.fori_loop(0, NPW + 1, init_row, 0)

        def start_gather(g, par, gsem):
            pltpu.async_copy(A_hbm.at[cidx.at[pl.ds(g * GRP, GRP)]],
                             rows.at[par], gsem)

        def wait_gather(g, par, gsem):
            pltpu.make_async_copy(A_hbm.at[cidx.at[pl.ds(g * GRP, GRP)]],
                                  rows.at[par], gsem).wait()

        iot0 = lax.iota(jnp.int32, LANES) * 0

        def accum_group(base, par):
            def edge(j, _):
                dl = cdst[pl.ds(base + j, LANES)][0]
                dsp = cdist[pl.ds(base + j, LANES)][iot0]
                dib = plsc.pack(dsp, dsp, format=plsc.PackFormat.INTERLEAVED)
                off = dl * DW

                @plsc.parallel_loop(0, NV, 1, unroll=NV)
                def _(v):
                    sl = pl.ds(off + v * LANES, LANES)
                    rw = plsc.bitcast(rows[par, j, pl.ds(v * LANES, LANES)],
                                      jnp.bfloat16)
                    wv = plsc.bitcast(wdv[pl.ds(v * LANES, LANES)],
                                      jnp.bfloat16)
                    av = plsc.bitcast(acc[sl], jnp.bfloat16)
                    val = jnp.minimum(av, rw + dib * wv)
                    acc[sl] = plsc.bitcast(val, jnp.int32)
                return 0
            lax.fori_loop(0, GRP, edge, 0)

        def run_groups(ngroups):
            @pl.when(ngroups > 0)
            def _():
                start_gather(0, 0, sg0)

            @pl.when(ngroups > 1)
            def _():
                start_gather(1, 1, sg1)

            npairs = ngroups // 2

            def pair(k, _):
                g0 = k * 2
                wait_gather(g0, 0, sg0)
                accum_group(g0 * GRP, 0)

                @pl.when(g0 + 2 < ngroups)
                def _():
                    start_gather(g0 + 2, 0, sg0)

                g1 = g0 + 1
                wait_gather(g1, 1, sg1)
                accum_group(g1 * GRP, 1)

                @pl.when(g1 + 2 < ngroups)
                def _():
                    start_gather(g1 + 2, 1, sg1)
                return 0
            lax.fori_loop(0, npairs, pair, 0)

            @pl.when(npairs * 2 < ngroups)
            def _():
                g = npairs * 2
                wait_gather(g, 0, sg0)
                accum_group(g * GRP, 0)

        NCH = EP // CH

        def load_chunk(c, par):
            e0 = c * CH
            b = pl.ds(par * CH, CH)
            pltpu.async_copy(dst_hbm.at[pl.ds(e0, CH)], dstb.at[b], semc)
            pltpu.async_copy(src_hbm.at[pl.ds(e0, CH)], srcb.at[b], semc)
            pltpu.async_copy(dist_hbm.at[pl.ds(e0, CH)], distb.at[b], semc)

        def wait_chunk(c, par):
            e0 = c * CH
            b = pl.ds(par * CH, CH)
            pltpu.make_async_copy(dst_hbm.at[pl.ds(e0, CH)], dstb.at[b],
                                  semc).wait()
            pltpu.make_async_copy(src_hbm.at[pl.ds(e0, CH)], srcb.at[b],
                                  semc).wait()
            pltpu.make_async_copy(dist_hbm.at[pl.ds(e0, CH)], distb.at[b],
                                  semc).wait()

        load_chunk(0, 0)

        lane15 = jnp.full((LANES,), LANES - 1, dtype=jnp.int32)

        def chunk(c, count_v):
            par = lax.rem(c, jnp.int32(2))
            wait_chunk(c, par)

            @pl.when(c + 1 < NCH)
            def _():
                load_chunk(c + 1, 1 - par)

            NS = CH // LANES
            iot = lax.iota(jnp.int32, LANES)
            l0 = iot == 0

            @plsc.parallel_loop(0, NS, 1, unroll=8)
            def _(i):
                s = pl.ds(par * CH + i * LANES, LANES)
                d16 = dstb[s]
                msk = (d16 >= lo) & (d16 < lo + NPW)
                pop = plsc.all_reduce_population_count(msk)
                plsc.store_scatter(baseb, [iot * 0 + i], pop, mask=l0)

            def prefix(k, carry):
                run_v, nz_v = carry
                sl = pl.ds(k * LANES, LANES)
                v = baseb[sl]
                csum = plsc.cumsum(v)
                baseb[sl] = run_v + csum - v
                nmsk = v > 0
                ncs = plsc.cumsum(jnp.where(nmsk, 1, 0))
                plsc.store_scatter(wlist, [nz_v + ncs - 1], k * LANES + iot,
                                   mask=nmsk)
                return run_v + csum[lane15], nz_v + ncs[lane15]
            count_v, nz_v = lax.fori_loop(
                0, NS // LANES, prefix,
                (count_v, jnp.zeros((LANES,), jnp.int32)))

            nz = nz_v[0]
            nzp = ((nz + 3) // 4) * 4

            @pl.when(nz % 4 != 0)
            def _():
                wlist[pl.ds(nz, LANES)] = jnp.zeros((LANES,), jnp.int32)

            @plsc.parallel_loop(0, nzp, 1, unroll=4)
            def _(k):
                i = wlist[pl.ds(k, LANES)][0]
                bvec = baseb[pl.ds(i, LANES)]
                base = bvec[iot * 0]
                s = pl.ds(par * CH + i * LANES, LANES)
                d16 = dstb[s]
                msk = (d16 >= lo) & (d16 < lo + NPW)
                csum = plsc.cumsum(jnp.where(msk, 1, 0))
                p = base + csum - 1
                plsc.store_scatter(cdst, [p], d16 - lo, mask=msk)
                plsc.store_scatter(cidx, [p], srcb[s], mask=msk)
                plsc.store_scatter(cdist, [p], distb[s], mask=msk)

            count = count_v[0]
            do_drain = (lax.rem(c, jnp.int32(DRAIN)) == DRAIN - 1) \
                | (c == NCH - 1)
            ngroups = jnp.where(do_drain, count // GRP, 0)
            run_groups(ngroups)

            @pl.when(ngroups > 0)
            def _():
                for k in range(GRP // LANES):
                    t = pl.ds(ngroups * GRP + k * LANES, LANES)
                    h = pl.ds(k * LANES, LANES)
                    cdst[h] = cdst[t]
                    cidx[h] = cidx[t]
                    cdist[h] = cdist[t]

            return jnp.full((LANES,), count - ngroups * GRP, dtype=jnp.int32)

        rem_v = lax.fori_loop(0, EP // CH, chunk,
                              jnp.zeros((LANES,), jnp.int32))
        rem = rem_v[0]

        @pl.when(rem > 0)
        def _():
            for k in range(GRP // LANES):
                p = pl.ds(rem + k * LANES, LANES)
                cidx[p] = jnp.zeros((LANES,), jnp.int32)
                cdst[p] = jnp.full((LANES,), NPW, dtype=jnp.int32)
                cdist[p] = jnp.zeros((LANES,), jnp.float32)
            start_gather(0, 0, sg0)
            wait_gather(0, 0, sg0)
            accum_group(0, 0)

        pltpu.sync_copy(acc.at[pl.ds(0, NPW * DW)],
                        m_hbm.at[pl.ds(lo * DW, NPW * DW)])

    return seg_kernel(A32, src, dst, dist, wd32)


def kernel(node_feat, dist, edge_index, W_edge, W_node):
    N, D = node_feat.shape
    O = W_edge.shape[0]
    E = dist.shape[0]

    Wu = W_edge[:, :D]
    wd = W_edge[:, D]
    Wv = W_edge[:, D + 1:]
    Wn1 = W_node[:, :D]
    Wn2 = W_node[:, D:]

    info = plsc.get_sparse_core_info()
    NW = info.num_cores * info.num_subcores
    NPW = -(-N // (NW * 64)) * 64
    NP = NW * NPW
    NP = -(-NP // ROW_BLK) * ROW_BLK
    NPW = NP // NW

    src = edge_index[0]
    dst = edge_index[1]
    EP = -(-E // CH) * CH
    if EP != E:
        src = jnp.pad(src, (0, EP - E))
        dist = jnp.pad(dist, (0, EP - E))
        dst = jnp.pad(dst, (0, EP - E), constant_values=-1)

    A32 = _proj(node_feat, Wu)
    wd16 = wd.astype(jnp.bfloat16)
    wd_lo = lax.bitcast_convert_type(wd16[:O // 2], jnp.uint16
                                     ).astype(jnp.int32)
    wd_hi = lax.bitcast_convert_type(wd16[O // 2:], jnp.uint16
                                     ).astype(jnp.int32)
    wd32 = wd_lo | (wd_hi << 16)
    m_flat = _segmin_call(A32, src, dst, dist, wd32, NP, O, NW, NPW)
    out = _final(node_feat, m_flat.reshape(NP, O // 2), Wv, Wn1, Wn2)
    return out
